# baseline (speedup 1.0000x reference)
import jax
import jax.numpy as jnp
from jax import lax
from jax.experimental import pallas as pl
from jax.experimental.pallas import tpu as pltpu
from jax.experimental.pallas import tpu_sc as plsc

_ORDER = 50
_B = 4096
_L = 200
_BB = 1024
_NW = 32
_BPW = _B // _NW
_LANES = 16
_NN = 100000
_ALIGN = (_NN // 128) * 128
_NPAD = _ALIGN + 128
_CHUNK = 6272
_TAILC = _ALIGN - 15 * _CHUNK


def _softplus(x):
    return jnp.maximum(x, 0.0) + jnp.log1p(jnp.exp(-jnp.abs(x)))


def _sc_body(src_hbm, dst_hbm, emb_hbm, tail_hbm, z_hbm,
             tstage, idx_s, idx_d, gs, gd, z_v,
             e0, e1, e2, e3, e4, e5, e6, e7, sem_s, sem_d):
    esh = (e0, e1, e2, e3, e4, e5, e6, e7)
    sid = lax.axis_index("s")
    base = (sid * 2 + lax.axis_index("c")) * _BPW
    pltpu.sync_copy(src_hbm.at[pl.ds(base, _BPW)], idx_s)
    pltpu.sync_copy(dst_hbm.at[pl.ds(base, _BPW)], idx_d)

    j0 = sid * _CHUNK

    @pl.when(sid < 15)
    def _():
        for c in range(8):
            pltpu.async_copy(emb_hbm.at[c, pl.ds(j0, _CHUNK)],
                             esh[c].at[pl.ds(j0, _CHUNK)], sem_s)
        for c in range(8):
            pltpu.make_async_copy(emb_hbm.at[c, pl.ds(j0, _CHUNK)],
                                  esh[c].at[pl.ds(j0, _CHUNK)], sem_s).wait()

    @pl.when(sid == 15)
    def _():
        cp1 = pltpu.async_copy(tail_hbm, tstage, sem_d)
        for c in range(8):
            pltpu.async_copy(emb_hbm.at[c, pl.ds(15 * _CHUNK, _TAILC)],
                             esh[c].at[pl.ds(15 * _CHUNK, _TAILC)], sem_s)
        cp1.wait()
        ntail = _NN - _ALIGN
        for c in range(8):
            pltpu.async_copy(tstage.at[c, pl.ds(128 - ntail, ntail)],
                             esh[c].at[pl.ds(_ALIGN, ntail)], sem_d)
        for c in range(8):
            pltpu.make_async_copy(emb_hbm.at[c, pl.ds(15 * _CHUNK, _TAILC)],
                                  esh[c].at[pl.ds(15 * _CHUNK, _TAILC)],
                                  sem_s).wait()
            pltpu.make_async_copy(tstage.at[c, pl.ds(128 - ntail, ntail)],
                                  esh[c].at[pl.ds(_ALIGN, ntail)], sem_d).wait()

    plsc.subcore_barrier()

    copies = []
    for c in range(8):
        copies.append(pltpu.async_copy(esh[c].at[idx_s], gs.at[c], sem_s))
        copies.append(pltpu.async_copy(esh[c].at[idx_d], gd.at[c], sem_d))
    for cp in copies:
        cp.wait()

    for k in range(_BPW // _LANES):
        sl = pl.ds(k * _LANES, _LANES)
        acc = gs[0, sl] * gd[0, sl]
        for c in range(1, 8):
            acc = acc + gs[c, sl] * gd[c, sl]
        z_v[sl] = acc
    pltpu.sync_copy(z_v, z_hbm.at[pl.ds(base, _BPW)])


def _sc_dot(src, dst, embT, emb_tail):
    mesh = plsc.VectorSubcoreMesh(core_axis_name="c", subcore_axis_name="s")
    return pl.kernel(
        _sc_body,
        out_type=jax.ShapeDtypeStruct((_B,), jnp.float32),
        mesh=mesh,
        scratch_types=[
            pltpu.VMEM((8, 128), jnp.float32),
            pltpu.VMEM((_BPW,), jnp.int32),
            pltpu.VMEM((_BPW,), jnp.int32),
            pltpu.VMEM((8, _BPW), jnp.float32),
            pltpu.VMEM((8, _BPW), jnp.float32),
            pltpu.VMEM((_BPW,), jnp.float32),
            *[pltpu.VMEM_SHARED((_NPAD,), jnp.float32) for _ in range(8)],
            pltpu.SemaphoreType.DMA,
            pltpu.SemaphoreType.DMA,
        ],
    )(src, dst, embT, emb_tail)


def _tc_body(tT_ref, xT_ref, t_ref, wa_ref, ba_ref, wb_ref, bb_ref,
             out_ref, m_ref, tri_ref):
    step = pl.program_id(0)

    @pl.when(step == 0)
    def _():
        mask = (tT_ref[...] < t_ref[...][None, :]).astype(jnp.float32)
        counts = jnp.sum(mask, axis=0)
        m_ref[0] = jnp.max(counts)
        li = lax.broadcasted_iota(jnp.int32, (_L, _L), 0)
        ki = lax.broadcasted_iota(jnp.int32, (_L, _L), 1)
        tri_ref[...] = (ki <= li).astype(jnp.bfloat16)

    @pl.when(step > 0)
    def _():
        b0 = (step - 1) * _BB
        tp = tT_ref[:, pl.ds(b0, _BB)]
        tt = t_ref[pl.ds(b0, _BB)]
        dt = tt[None, :] - tp
        mask = dt > 0.0
        maskf = mask.astype(jnp.bfloat16)
        macc = jax.lax.dot(tri_ref[...], maskf,
                           preferred_element_type=jnp.float32)
        keep = mask & (macc > m_ref[0] - _ORDER)

        wa0, wa1, wa2 = wa_ref[0, 0], wa_ref[0, 1], wa_ref[0, 2]
        wb0, wb1, wb2 = wb_ref[0, 0], wb_ref[0, 1], wb_ref[0, 2]
        x0 = xT_ref[0, :, :]
        x1 = xT_ref[1, :, :]
        x2 = xT_ref[2, :, :]
        a_lin = x0 * wa0 + x1 * wa1 + x2 * wa2 + ba_ref[0]
        b_lin = x0 * wb0 + x1 * wb1 + x2 * wb2 + bb_ref[0]
        alphas = jnp.log1p(jnp.exp(a_lin))
        betas = jnp.log1p(jnp.exp(b_lin))
        terms = jnp.where(keep, alphas * jnp.exp(-betas * dt), 0.0)
        out_ref[...] = jnp.sum(terms, axis=0)


def _tc_dense(tT, xT, t, wa, ba, wb, bb):
    grid = (_B // _BB + 1,)
    return pl.pallas_call(
        _tc_body,
        grid=grid,
        in_specs=[
            pl.BlockSpec((_L, _B), lambda i: (0, 0)),
            pl.BlockSpec((3, _L, _BB), lambda i: (0, 0, jnp.maximum(i - 1, 0))),
            pl.BlockSpec((_B,), lambda i: (0,)),
            pl.BlockSpec(memory_space=pltpu.SMEM),
            pl.BlockSpec(memory_space=pltpu.SMEM),
            pl.BlockSpec(memory_space=pltpu.SMEM),
            pl.BlockSpec(memory_space=pltpu.SMEM),
        ],
        out_specs=pl.BlockSpec((_BB,), lambda i: (jnp.maximum(i - 1, 0),)),
        out_shape=jax.ShapeDtypeStruct((_B,), jnp.float32),
        scratch_shapes=[pltpu.SMEM((1,), jnp.float32),
                        pltpu.VMEM((_L, _L), jnp.bfloat16)],
    )(tT, xT, t, wa, ba, wb, bb)


def _combine_body(z_ref, incr_ref, out_ref):
    out_ref[...] = _softplus(z_ref[...]) + incr_ref[...]


def _combine(z, incr):
    return pl.pallas_call(
        _combine_body,
        out_shape=jax.ShapeDtypeStruct((_B,), jnp.float32),
    )(z, incr)


def kernel(src, dst, t, x_pad, t_pad, emb, W_alpha, b_alpha, W_beta, b_beta):
    xT = jnp.transpose(x_pad, (2, 1, 0))
    tT = jnp.transpose(t_pad, (1, 0))
    embT = jnp.transpose(emb, (1, 0))
    emb_tail = lax.dynamic_slice(embT, (0, _NN - 128), (8, 128))
    z = _sc_dot(src.astype(jnp.int32), dst.astype(jnp.int32), embT, emb_tail)
    incr = _tc_dense(tT, xT, t, W_alpha, b_alpha, W_beta, b_beta)
    return _combine(z, incr)


# speedup vs baseline: 1.0052x; 1.0052x over previous
import jax
import jax.numpy as jnp
from jax import lax
from jax.experimental import pallas as pl
from jax.experimental.pallas import tpu as pltpu
from jax.experimental.pallas import tpu_sc as plsc

_ORDER = 50
_B = 4096
_L = 200
_BB = 1024
_NW = 32
_BPW = _B // _NW
_LANES = 16
_NN = 100000
_ALIGN = (_NN // 128) * 128
_NPAD = _ALIGN + 128
_CHUNK = 6272
_TAILC = _ALIGN - 15 * _CHUNK


def _softplus(x):
    return jnp.maximum(x, 0.0) + jnp.log1p(jnp.exp(-jnp.abs(x)))


def _sc_body(src_hbm, dst_hbm, emb_hbm, tail_hbm, z_hbm,
             tstage, idx_s, idx_d, gs, gd, z_v,
             e0, e1, e2, e3, e4, e5, e6, e7, sem_s, sem_d):
    esh = (e0, e1, e2, e3, e4, e5, e6, e7)
    sid = lax.axis_index("s")
    base = (sid * 2 + lax.axis_index("c")) * _BPW
    pltpu.sync_copy(src_hbm.at[pl.ds(base, _BPW)], idx_s)
    pltpu.sync_copy(dst_hbm.at[pl.ds(base, _BPW)], idx_d)

    j0 = sid * _CHUNK

    @pl.when(sid < 15)
    def _():
        for c in range(8):
            pltpu.async_copy(emb_hbm.at[c, pl.ds(j0, _CHUNK)],
                             esh[c].at[pl.ds(j0, _CHUNK)], sem_s)
        for c in range(8):
            pltpu.make_async_copy(emb_hbm.at[c, pl.ds(j0, _CHUNK)],
                                  esh[c].at[pl.ds(j0, _CHUNK)], sem_s).wait()

    @pl.when(sid == 15)
    def _():
        cp1 = pltpu.async_copy(tail_hbm, tstage, sem_d)
        for c in range(8):
            pltpu.async_copy(emb_hbm.at[c, pl.ds(15 * _CHUNK, _TAILC)],
                             esh[c].at[pl.ds(15 * _CHUNK, _TAILC)], sem_s)
        cp1.wait()
        for c in range(8):
            pltpu.async_copy(tstage.at[c], esh[c].at[pl.ds(_ALIGN, 128)], sem_d)
        for c in range(8):
            pltpu.make_async_copy(emb_hbm.at[c, pl.ds(15 * _CHUNK, _TAILC)],
                                  esh[c].at[pl.ds(15 * _CHUNK, _TAILC)],
                                  sem_s).wait()
            pltpu.make_async_copy(tstage.at[c], esh[c].at[pl.ds(_ALIGN, 128)],
                                  sem_d).wait()

    plsc.subcore_barrier()

    copies = []
    for c in range(8):
        copies.append(pltpu.async_copy(esh[c].at[idx_s], gs.at[c], sem_s))
        copies.append(pltpu.async_copy(esh[c].at[idx_d], gd.at[c], sem_d))
    for cp in copies:
        cp.wait()

    for k in range(_BPW // _LANES):
        sl = pl.ds(k * _LANES, _LANES)
        acc = gs[0, sl] * gd[0, sl]
        for c in range(1, 8):
            acc = acc + gs[c, sl] * gd[c, sl]
        z_v[sl] = acc
    pltpu.sync_copy(z_v, z_hbm.at[pl.ds(base, _BPW)])


def _sc_dot(src, dst, embT, emb_tail):
    mesh = plsc.VectorSubcoreMesh(core_axis_name="c", subcore_axis_name="s")
    return pl.kernel(
        _sc_body,
        out_type=jax.ShapeDtypeStruct((_B,), jnp.float32),
        mesh=mesh,
        scratch_types=[
            pltpu.VMEM((8, 128), jnp.float32),
            pltpu.VMEM((_BPW,), jnp.int32),
            pltpu.VMEM((_BPW,), jnp.int32),
            pltpu.VMEM((8, _BPW), jnp.float32),
            pltpu.VMEM((8, _BPW), jnp.float32),
            pltpu.VMEM((_BPW,), jnp.float32),
            *[pltpu.VMEM_SHARED((_NPAD,), jnp.float32) for _ in range(8)],
            pltpu.SemaphoreType.DMA,
            pltpu.SemaphoreType.DMA,
        ],
    )(src, dst, embT, emb_tail)


def _tc_body(tT_ref, xT_ref, t_ref, wa_ref, ba_ref, wb_ref, bb_ref,
             out_ref, m_ref, tri_ref):
    step = pl.program_id(0)

    @pl.when(step == 0)
    def _():
        mask = (tT_ref[...] < t_ref[...][None, :]).astype(jnp.float32)
        counts = jnp.sum(mask, axis=0)
        m_ref[0] = jnp.max(counts)
        li = lax.broadcasted_iota(jnp.int32, (_L, _L), 0)
        ki = lax.broadcasted_iota(jnp.int32, (_L, _L), 1)
        tri_ref[...] = (ki <= li).astype(jnp.bfloat16)

    @pl.when(step > 0)
    def _():
        b0 = (step - 1) * _BB
        tp = tT_ref[:, pl.ds(b0, _BB)]
        tt = t_ref[pl.ds(b0, _BB)]
        dt = tt[None, :] - tp
        mask = dt > 0.0
        maskf = mask.astype(jnp.bfloat16)
        macc = jax.lax.dot(tri_ref[...], maskf,
                           preferred_element_type=jnp.float32)
        keep = mask & (macc > m_ref[0] - _ORDER)

        wa0, wa1, wa2 = wa_ref[0, 0], wa_ref[0, 1], wa_ref[0, 2]
        wb0, wb1, wb2 = wb_ref[0, 0], wb_ref[0, 1], wb_ref[0, 2]
        x0 = xT_ref[0, :, :]
        x1 = xT_ref[1, :, :]
        x2 = xT_ref[2, :, :]
        a_lin = x0 * wa0 + x1 * wa1 + x2 * wa2 + ba_ref[0]
        b_lin = x0 * wb0 + x1 * wb1 + x2 * wb2 + bb_ref[0]
        alphas = jnp.log1p(jnp.exp(a_lin))
        betas = jnp.log1p(jnp.exp(b_lin))
        terms = jnp.where(keep, alphas * jnp.exp(-betas * dt), 0.0)
        out_ref[...] = jnp.sum(terms, axis=0)


def _tc_dense(tT, xT, t, wa, ba, wb, bb):
    grid = (_B // _BB + 1,)
    return pl.pallas_call(
        _tc_body,
        grid=grid,
        in_specs=[
            pl.BlockSpec((_L, _B), lambda i: (0, 0)),
            pl.BlockSpec((3, _L, _BB), lambda i: (0, 0, jnp.maximum(i - 1, 0))),
            pl.BlockSpec((_B,), lambda i: (0,)),
            pl.BlockSpec(memory_space=pltpu.SMEM),
            pl.BlockSpec(memory_space=pltpu.SMEM),
            pl.BlockSpec(memory_space=pltpu.SMEM),
            pl.BlockSpec(memory_space=pltpu.SMEM),
        ],
        out_specs=pl.BlockSpec((_BB,), lambda i: (jnp.maximum(i - 1, 0),)),
        out_shape=jax.ShapeDtypeStruct((_B,), jnp.float32),
        scratch_shapes=[pltpu.SMEM((1,), jnp.float32),
                        pltpu.VMEM((_L, _L), jnp.bfloat16)],
    )(tT, xT, t, wa, ba, wb, bb)


def _combine_body(z_ref, incr_ref, out_ref):
    out_ref[...] = _softplus(z_ref[...]) + incr_ref[...]


def _combine(z, incr):
    return pl.pallas_call(
        _combine_body,
        out_shape=jax.ShapeDtypeStruct((_B,), jnp.float32),
    )(z, incr)


def kernel(src, dst, t, x_pad, t_pad, emb, W_alpha, b_alpha, W_beta, b_beta):
    xT = jnp.transpose(x_pad, (2, 1, 0))
    tT = jnp.transpose(t_pad, (1, 0))
    embT = jnp.transpose(emb, (1, 0))
    emb_tail = jnp.pad(embT[:, _ALIGN:], ((0, 0), (0, 128 - (_NN - _ALIGN))))
    z = _sc_dot(src.astype(jnp.int32), dst.astype(jnp.int32), embT, emb_tail)
    incr = _tc_dense(tT, xT, t, W_alpha, b_alpha, W_beta, b_beta)
    return _combine(z, incr)
